# baseline (speedup 1.0000x reference)
import jax
import jax.numpy as jnp
from jax import lax
from jax.experimental import pallas as pl

_NH = 4
_NPNT = 4


def _body(x1_ref, x2_ref, qpos_ref, ln1w_ref, ln1b_ref, ln2w_ref, ln2b_ref,
          sow_ref, sob_ref, aww_ref, awb_ref, vpw_ref, vpb_ref,
          opw_ref, opb_ref, out_ref):
    C, nq = x1_ref.shape[1], x1_ref.shape[2]
    W = 32
    hd = C // _NH

    x1b = x1_ref[0]
    x2b = x2_ref[0]

    def ln_t(x, w, b):
        mu = jnp.mean(x, axis=0, keepdims=True)
        xc = x - mu
        var = jnp.mean(xc * xc, axis=0, keepdims=True)
        return xc * lax.rsqrt(var + 1e-5) * w + b

    queryT = ln_t(x1b, ln1w_ref[...], ln1b_ref[...]) + qpos_ref[...]
    valueT = ln_t(x2b, ln2w_ref[...], ln2b_ref[...])

    vT = lax.dot_general(vpw_ref[...], valueT, (((0,), (0,)), ((), ())),
                         preferred_element_type=jnp.float32) + vpb_ref[...]
    soT = lax.dot_general(sow_ref[...], queryT, (((0,), (0,)), ((), ())),
                          preferred_element_type=jnp.float32) + sob_ref[...]
    awT = lax.dot_general(aww_ref[...], queryT, (((0,), (0,)), ((), ())),
                          preferred_element_type=jnp.float32) + awb_ref[...]

    qi = lax.broadcasted_iota(jnp.int32, (1, nq), 1)
    colq = (qi % W).astype(jnp.float32)
    rowq = (qi // W).astype(jnp.float32)
    xg = lax.broadcasted_iota(jnp.int32, (W, nq), 0).astype(jnp.float32)

    outsT = []
    for h in range(_NH):
        rows = [awT[h * _NPNT + p:h * _NPNT + p + 1, :] for p in range(_NPNT)]
        m = jnp.maximum(jnp.maximum(rows[0], rows[1]), jnp.maximum(rows[2], rows[3]))
        es = [jnp.exp(r - m) for r in rows]
        inv = 1.0 / (es[0] + es[1] + es[2] + es[3])

        at3 = None
        for p in range(_NPNT):
            o = (h * _NPNT + p) * 2
            x = colq + soT[o:o + 1, :]
            y = rowq + soT[o + 1:o + 2, :]
            wx = jnp.maximum(1.0 - jnp.abs(x - xg), 0.0)
            wy = jnp.maximum(1.0 - jnp.abs(y - xg), 0.0)
            wxa = wx * (es[p] * inv)
            term = wy[:, None, :] * wxa[None, :, :]
            at3 = term if at3 is None else at3 + term
        atm = at3.reshape(nq, nq)
        v_hT = vT[h * hd:(h + 1) * hd, :]
        out_hT = jnp.dot(v_hT, atm, preferred_element_type=jnp.float32)
        outsT.append(out_hT)

    sampledT = jnp.concatenate(outsT, axis=0)
    finalT = lax.dot_general(opw_ref[...], sampledT, (((0,), (0,)), ((), ())),
                             preferred_element_type=jnp.float32)
    out_ref[0] = finalT + opb_ref[...] + x2b


def kernel(x1, x2, ln1_w, ln1_b, ln2_w, ln2_b, pos_scale, so_w, so_b,
           aw_w, aw_b, vp_w, vp_b, op_w, op_b):
    B, C, H, W = x1.shape
    nq = H * W

    x1t = x1.reshape(B, C, nq)
    x2t = x2.reshape(B, C, nq)

    inv_freq = 1.0 / (10000.0 ** (jnp.arange(0, C, 2, dtype=jnp.float32) / C))
    t = jnp.arange(nq, dtype=jnp.float32)
    sinu = inv_freq[:, None] * t[None, :]
    qposT = jnp.concatenate([jnp.sin(sinu), jnp.cos(sinu)], axis=0) * pos_scale

    full = lambda shape: pl.BlockSpec(shape, lambda b: (0,) * len(shape))
    out = pl.pallas_call(
        _body,
        grid=(B,),
        in_specs=[
            pl.BlockSpec((1, C, nq), lambda b: (b, 0, 0)),
            pl.BlockSpec((1, C, nq), lambda b: (b, 0, 0)),
            full((C, nq)),
            full((C, 1)), full((C, 1)), full((C, 1)), full((C, 1)),
            full((C, _NH * _NPNT * 2)), full((_NH * _NPNT * 2, 1)),
            full((C, _NH * _NPNT)), full((_NH * _NPNT, 1)),
            full((C, C)), full((C, 1)),
            full((C, C)), full((C, 1)),
        ],
        out_specs=pl.BlockSpec((1, C, nq), lambda b: (b, 0, 0)),
        out_shape=jax.ShapeDtypeStruct((B, C, nq), jnp.float32),
    )(x1t, x2t, qposT,
      ln1_w.reshape(C, 1), ln1_b.reshape(C, 1), ln2_w.reshape(C, 1), ln2_b.reshape(C, 1),
      so_w, so_b.reshape(-1, 1), aw_w, aw_b.reshape(-1, 1),
      vp_w, vp_b.reshape(C, 1), op_w, op_b.reshape(C, 1))
    return out.reshape(B, C, H, W)
